# Initial kernel scaffold; baseline (speedup 1.0000x reference)
import jax
import jax.numpy as jnp
from jax.experimental import pallas as pl


def kernel(x, edge_index, W1, a_src1, a_dst1, b1, W2, a_src2, a_dst2, b2, W3, a_src3, a_dst3, b3):
    raise NotImplementedError("write your pallas kernel here")


# speedup vs baseline: 112.5770x; 112.5770x over previous
import functools

import jax
import jax.numpy as jnp
from jax import lax
from jax.experimental import pallas as pl
from jax.experimental.pallas import tpu as pltpu
from jax.experimental.pallas import tpu_sc as plsc

N = 50000
HID = 6
E = 1600000
LANES = 128
NR = 391
NP = NR * LANES
NW = 32
CH = 1024
NCHUNK = E // CH
TPW = NCHUNK // NW
ROWS_PT = NP // 16
ZROWS = 782

_F32 = jnp.float32


def _node_mask():
    r = lax.broadcasted_iota(jnp.int32, (NR, LANES), 0)
    c = lax.broadcasted_iota(jnp.int32, (NR, LANES), 1)
    return r * LANES + c < N


def _leaky(v):
    return jnp.maximum(v, 0.2 * v)


def _d0_body(x_ref, w1_ref, asr_ref, adr_ref, hst_ref, acol_ref):
    x = x_ref[...]
    for c in range(HID):
        hst_ref[c] = x * w1_ref[0, c]
    cs = sum(w1_ref[0, k] * asr_ref[k] for k in range(HID))
    cd = sum(w1_ref[0, k] * adr_ref[k] for k in range(HID))
    asv = x * cs
    adv = x * cd
    hst_ref[HID] = asv
    hst_ref[HID + 1] = adv
    amax = jnp.max(jnp.where(_node_mask(), asv, -jnp.inf))
    acol_ref[...] = jnp.full((NR, LANES), amax, _F32)


def _make_fin_body(cin, cout):
    def body(aa_ref, ab_ref, hst_ref, acol_ref, w_ref, asr_ref, adr_ref,
             b_ref, ohst_ref, oacol_ref):
        asv = hst_ref[cin]
        adv = hst_ref[cin + 1]
        wself = jnp.exp(_leaky(asv + adv) - _leaky(adv + acol_ref[...]))
        den = aa_ref[cin + 1] + ab_ref[cin + 1] + wself
        inv = 1.0 / (den + 1e-16)
        xs = []
        for c in range(cin):
            num = aa_ref[c] + ab_ref[c] + wself * hst_ref[c]
            xs.append(jnp.maximum(num * inv + b_ref[c], 0.0))
        hn = []
        for c in range(cout):
            acc = xs[0] * w_ref[0, c]
            for k in range(1, cin):
                acc = acc + xs[k] * w_ref[k, c]
            hn.append(acc)
            ohst_ref[c] = acc
        asn = hn[0] * asr_ref[0]
        adn = hn[0] * adr_ref[0]
        for c in range(1, cout):
            asn = asn + hn[c] * asr_ref[c]
            adn = adn + hn[c] * adr_ref[c]
        ohst_ref[cout] = asn
        ohst_ref[cout + 1] = adn
        amax = jnp.max(jnp.where(_node_mask(), asn, -jnp.inf))
        oacol_ref[...] = jnp.full((NR, LANES), amax, _F32)
    return body


def _d3_body(aa_ref, ab_ref, hst_ref, acol_ref, b_ref, out_ref):
    asv = hst_ref[1]
    adv = hst_ref[2]
    wself = jnp.exp(_leaky(asv + adv) - _leaky(adv + acol_ref[...]))
    den = aa_ref[2] + ab_ref[2] + wself
    z = (aa_ref[0] + ab_ref[0] + wself * hst_ref[0]) / (den + 1e-16) + b_ref[0]
    mask = _node_mask()
    zmax = jnp.max(jnp.where(mask, z, -jnp.inf))
    ez = jnp.where(mask, jnp.exp(z - zmax), 0.0)
    out_ref[...] = ez / jnp.sum(ez)


_SMEM = pl.BlockSpec(memory_space=pltpu.MemorySpace.SMEM)
_VSPEC = pl.BlockSpec(memory_space=pltpu.MemorySpace.VMEM)


def _tc_call(body, n_in_vmem, n_in_smem, out_shapes):
    return pl.pallas_call(
        body,
        in_specs=[_VSPEC] * n_in_vmem + [_SMEM] * n_in_smem,
        out_specs=[_VSPEC] * len(out_shapes),
        out_shape=[jax.ShapeDtypeStruct(s, _F32) for s in out_shapes],
    )


def _sc_edge_body(rec_hbm, ad_hbm, av_hbm, srcv_hbm, dstv_hbm,
                  acca_hbm, accb_hbm,
                  srcidx, dstidx, hrow, adbuf, wbuf, a16, zbuf, acc_sh,
                  sem_g, sem_a):
    cid = lax.axis_index("c")
    sid = lax.axis_index("s")
    wid = sid * 2 + cid

    pltpu.sync_copy(av_hbm, a16)
    av = a16[...]
    iota = lax.iota(jnp.int32, 16)
    dv8 = iota // 8
    md8 = iota % 8
    col6 = jnp.full((16,), HID, jnp.int32)
    zero16 = jnp.zeros((16,), _F32)

    def zb(j, carry):
        plsc.store_scatter(zbuf, [2 * j + dv8, md8], zero16)
        return carry
    lax.fori_loop(0, ZROWS // 2 * 8 // 8, zb, 0)

    row0 = sid * ROWS_PT
    for q in range(4):
        pltpu.sync_copy(zbuf, acc_sh.at[pl.ds(row0 + q * ZROWS, ZROWS)])
    plsc.subcore_barrier()

    def chunk(t, carry):
        g = wid + NW * t
        pltpu.sync_copy(srcv_hbm.at[pl.ds(g * 8, 8)], srcidx)
        pltpu.sync_copy(dstv_hbm.at[pl.ds(g * 8, 8)], dstidx)
        copies = []
        for j in range(8):
            copies.append(pltpu.async_copy(
                rec_hbm.at[srcidx.at[j]],
                hrow.at[pl.ds(j * LANES, LANES)], sem_g))
            copies.append(pltpu.async_copy(
                ad_hbm.at[dstidx.at[j]],
                adbuf.at[pl.ds(j * LANES, LANES)], sem_a))
        for cdesc in copies:
            cdesc.wait()

        def wcomp(k, c2):
            e16 = k * 16 + iota
            asv = plsc.load_gather(hrow, [e16, col6])
            adv = plsc.load_gather(adbuf, [e16])
            s = asv + adv
            e = jnp.maximum(s, 0.2 * s)
            tt = adv + av
            mp = jnp.maximum(tt, 0.2 * tt)
            plsc.store_scatter(wbuf, [e16], jnp.exp(e - mp))
            return c2
        lax.fori_loop(0, CH // 16, wcomp, 0)

        def emul(k, c2):
            for p in range(8):
                r16 = (k * 16 + 2 * p) + dv8
                wv = plsc.load_gather(wbuf, [r16])
                hv = plsc.load_gather(hrow, [r16, md8])
                plsc.store_scatter(hrow, [r16, md8], wv * hv)
            return c2
        lax.fori_loop(0, CH // 16, emul, 0)

        for j in range(8):
            pltpu.sync_copy(hrow.at[pl.ds(j * LANES, LANES)],
                            acc_sh.at[dstidx.at[j]], add=True)
        return carry
    lax.fori_loop(0, TPW, chunk, 0)
    plsc.subcore_barrier()

    @pl.when(cid == 0)
    def _():
        pltpu.sync_copy(acc_sh.at[pl.ds(row0, ROWS_PT)],
                        acca_hbm.at[pl.ds(row0, ROWS_PT)])

    @pl.when(cid == 1)
    def _():
        pltpu.sync_copy(acc_sh.at[pl.ds(row0, ROWS_PT)],
                        accb_hbm.at[pl.ds(row0, ROWS_PT)])


_sc_edge = functools.partial(
    pl.kernel,
    out_type=(jax.ShapeDtypeStruct((NP, 8), _F32),
              jax.ShapeDtypeStruct((NP, 8), _F32)),
    mesh=plsc.VectorSubcoreMesh(core_axis_name="c", subcore_axis_name="s",
                                num_cores=2, num_subcores=16),
    compiler_params=pltpu.CompilerParams(needs_layout_passes=False,
                                         use_tc_tiling_on_sc=False),
    scratch_types=[
        pltpu.VMEM((8, LANES), jnp.int32),
        pltpu.VMEM((8, LANES), jnp.int32),
        pltpu.VMEM((CH, 8), _F32),
        pltpu.VMEM((CH,), _F32),
        pltpu.VMEM((CH,), _F32),
        pltpu.VMEM((16,), _F32),
        pltpu.VMEM((ZROWS, 8), _F32),
        pltpu.VMEM_SHARED((NP, 8), _F32),
        pltpu.SemaphoreType.DMA,
        pltpu.SemaphoreType.DMA,
    ],
)(_sc_edge_body)


def _cols(acc):
    return jnp.moveaxis(acc.reshape(NR, LANES, 8), 2, 0)


def _assemble_rec(hst, cfeat):
    ones = jnp.ones((1, NR, LANES), _F32)
    parts = [hst[:cfeat], hst[cfeat:cfeat + 1], ones]
    pad = 8 - (cfeat + 2)
    if pad:
        parts.append(jnp.zeros((pad, NR, LANES), _F32))
    return jnp.moveaxis(jnp.concatenate(parts, 0), 0, 2).reshape(NP, 8)


def kernel(x, edge_index, W1, a_src1, a_dst1, b1, W2, a_src2, a_dst2, b2,
           W3, a_src3, a_dst3, b3):
    xp = jnp.pad(x[:, 0], (0, NP - N)).reshape(NR, LANES)
    srcv = edge_index[0].reshape(E // LANES, LANES)
    dstv = edge_index[1].reshape(E // LANES, LANES)

    d0 = _tc_call(_d0_body, 1, 3, [(HID + 2, NR, LANES), (NR, LANES)])
    hst1, a1 = d0(xp, W1, a_src1, a_dst1)

    acc1a, acc1b = _sc_edge(_assemble_rec(hst1, HID),
                            hst1[HID + 1].reshape(NP),
                            a1.reshape(NP)[:16], srcv, dstv)

    fin12 = _tc_call(_make_fin_body(HID, HID), 4, 4,
                     [(HID + 2, NR, LANES), (NR, LANES)])
    hst2, a2 = fin12(_cols(acc1a), _cols(acc1b), hst1, a1,
                     W2, a_src2, a_dst2, b1)

    acc2a, acc2b = _sc_edge(_assemble_rec(hst2, HID),
                            hst2[HID + 1].reshape(NP),
                            a2.reshape(NP)[:16], srcv, dstv)

    fin23 = _tc_call(_make_fin_body(HID, 1), 4, 4,
                     [(3, NR, LANES), (NR, LANES)])
    hst3, a3 = fin23(_cols(acc2a), _cols(acc2b), hst2, a2,
                     W3, a_src3, a_dst3, b2)

    acc3a, acc3b = _sc_edge(_assemble_rec(hst3, 1),
                            hst3[2].reshape(NP),
                            a3.reshape(NP)[:16], srcv, dstv)

    d3 = _tc_call(_d3_body, 4, 1, [(NR, LANES)])
    out = d3(_cols(acc3a), _cols(acc3b), hst3, a3, b3)

    return out[0].reshape(NP)[:N][:, None]


# speedup vs baseline: 149.2411x; 1.3257x over previous
import functools

import jax
import jax.numpy as jnp
from jax import lax
from jax.experimental import pallas as pl
from jax.experimental.pallas import tpu as pltpu
from jax.experimental.pallas import tpu_sc as plsc

N = 50000
HID = 6
E = 1600000
LANES = 128
NR = 391
NP = NR * LANES
NW = 32
CH = 1024
NCHUNK = E // CH
TPW = NCHUNK // NW
ROWS_PT = NP // 16
ZROWS = 782

_F32 = jnp.float32


def _node_mask():
    r = lax.broadcasted_iota(jnp.int32, (NR, LANES), 0)
    c = lax.broadcasted_iota(jnp.int32, (NR, LANES), 1)
    return r * LANES + c < N


def _leaky(v):
    return jnp.maximum(v, 0.2 * v)


def _d0_body(x_ref, w1_ref, asr_ref, adr_ref, hst_ref, acol_ref):
    x = x_ref[...]
    for c in range(HID):
        hst_ref[c] = x * w1_ref[0, c]
    cs = sum(w1_ref[0, k] * asr_ref[k] for k in range(HID))
    cd = sum(w1_ref[0, k] * adr_ref[k] for k in range(HID))
    asv = x * cs
    adv = x * cd
    hst_ref[HID] = asv
    hst_ref[HID + 1] = adv
    amax = jnp.max(jnp.where(_node_mask(), asv, -jnp.inf))
    acol_ref[...] = jnp.full((NR, LANES), amax, _F32)


def _make_fin_body(cin, cout):
    def body(aa_ref, ab_ref, hst_ref, acol_ref, w_ref, asr_ref, adr_ref,
             b_ref, ohst_ref, oacol_ref):
        asv = hst_ref[cin]
        adv = hst_ref[cin + 1]
        wself = jnp.exp(_leaky(asv + adv) - _leaky(adv + acol_ref[...]))
        den = aa_ref[cin + 1] + ab_ref[cin + 1] + wself
        inv = 1.0 / (den + 1e-16)
        xs = []
        for c in range(cin):
            num = aa_ref[c] + ab_ref[c] + wself * hst_ref[c]
            xs.append(jnp.maximum(num * inv + b_ref[c], 0.0))
        hn = []
        for c in range(cout):
            acc = xs[0] * w_ref[0, c]
            for k in range(1, cin):
                acc = acc + xs[k] * w_ref[k, c]
            hn.append(acc)
            ohst_ref[c] = acc
        asn = hn[0] * asr_ref[0]
        adn = hn[0] * adr_ref[0]
        for c in range(1, cout):
            asn = asn + hn[c] * asr_ref[c]
            adn = adn + hn[c] * adr_ref[c]
        ohst_ref[cout] = asn
        ohst_ref[cout + 1] = adn
        amax = jnp.max(jnp.where(_node_mask(), asn, -jnp.inf))
        oacol_ref[...] = jnp.full((NR, LANES), amax, _F32)
    return body


def _d3_body(aa_ref, ab_ref, hst_ref, acol_ref, b_ref, out_ref):
    asv = hst_ref[1]
    adv = hst_ref[2]
    wself = jnp.exp(_leaky(asv + adv) - _leaky(adv + acol_ref[...]))
    den = aa_ref[2] + ab_ref[2] + wself
    z = (aa_ref[0] + ab_ref[0] + wself * hst_ref[0]) / (den + 1e-16) + b_ref[0]
    mask = _node_mask()
    zmax = jnp.max(jnp.where(mask, z, -jnp.inf))
    ez = jnp.where(mask, jnp.exp(z - zmax), 0.0)
    out_ref[...] = ez / jnp.sum(ez)


_SMEM = pl.BlockSpec(memory_space=pltpu.MemorySpace.SMEM)
_VSPEC = pl.BlockSpec(memory_space=pltpu.MemorySpace.VMEM)


def _tc_call(body, n_in_vmem, n_in_smem, out_shapes):
    return pl.pallas_call(
        body,
        in_specs=[_VSPEC] * n_in_vmem + [_SMEM] * n_in_smem,
        out_specs=[_VSPEC] * len(out_shapes),
        out_shape=[jax.ShapeDtypeStruct(s, _F32) for s in out_shapes],
    )


def _sc_edge_body(rec_hbm, ad_hbm, av_hbm, srcv_hbm, dstv_hbm,
                  acca_hbm, accb_hbm,
                  srcidx0, srcidx1, dstidx0, dstidx1, sdix0, sdix1,
                  hrow0, hrow1, orow0, orow1, adbuf0, adbuf1,
                  wbuf, a16, zbuf, acc_sh,
                  sg0, sg1, sa0, sa1, ss0, ss1):
    srcidx = (srcidx0, srcidx1)
    dstidx = (dstidx0, dstidx1)
    sdix = (sdix0, sdix1)
    hrow = (hrow0, hrow1)
    orow = (orow0, orow1)
    adbuf = (adbuf0, adbuf1)
    sg = (sg0, sg1)
    sa = (sa0, sa1)
    ss = (ss0, ss1)

    cid = lax.axis_index("c")
    sid = lax.axis_index("s")
    wid = sid * 2 + cid

    pltpu.sync_copy(av_hbm, a16)
    av = a16[...]
    iota = lax.iota(jnp.int32, 16)
    dv8 = iota // 8
    md8 = iota % 8
    col6 = jnp.full((16,), HID, jnp.int32)
    zero16 = jnp.zeros((16,), _F32)

    def zb(j, carry):
        plsc.store_scatter(zbuf, [2 * j + dv8, md8], zero16)
        return carry
    lax.fori_loop(0, ZROWS * 8 // 16, zb, 0)

    row0 = sid * ROWS_PT
    for q in range(4):
        pltpu.sync_copy(zbuf, acc_sh.at[pl.ds(row0 + q * ZROWS, ZROWS)])
    plsc.subcore_barrier()

    def issue(b, t):
        g = wid + NW * t
        pltpu.sync_copy(srcv_hbm.at[pl.ds(g * 8, 8)], srcidx[b])
        pltpu.sync_copy(dstv_hbm.at[pl.ds(g * 8, 8)], dstidx[b])
        for j in range(8):
            pltpu.async_copy(rec_hbm.at[srcidx[b].at[j]],
                             hrow[b].at[pl.ds(j * LANES, LANES)], sg[b])
            pltpu.async_copy(ad_hbm.at[dstidx[b].at[j]],
                             adbuf[b].at[pl.ds(j * LANES, LANES)], sa[b])

    for b in range(2):
        issue(b, b)

    def step(u, carry):
        for b in range(2):
            t = 2 * u + b
            pltpu.make_async_copy(rec_hbm.at[pl.ds(0, CH)], hrow[b],
                                  sg[b]).wait()
            pltpu.make_async_copy(ad_hbm.at[pl.ds(0, CH)], adbuf[b],
                                  sa[b]).wait()

            @pl.when(u >= 1)
            def _():
                pltpu.make_async_copy(rec_hbm.at[pl.ds(0, CH)], orow[b],
                                      ss[b]).wait()

            def wcomp(k, c2):
                e16 = k * 16 + iota
                asv = plsc.load_gather(hrow[b], [e16, col6])
                adv = plsc.load_gather(adbuf[b], [e16])
                s = asv + adv
                e = jnp.maximum(s, 0.2 * s)
                tt = adv + av
                mp = jnp.maximum(tt, 0.2 * tt)
                plsc.store_scatter(wbuf, [e16], jnp.exp(e - mp))
                r7 = e16 >> 7
                c7 = e16 & 127
                iv = plsc.load_gather(dstidx[b], [r7, c7])
                plsc.store_scatter(sdix[b], [r7, c7], iv)
                return c2
            lax.fori_loop(0, CH // 16, wcomp, 0)

            def emul(k, c2):
                for p in range(8):
                    r16 = (k * 16 + 2 * p) + dv8
                    wv = plsc.load_gather(wbuf, [r16])
                    hv = plsc.load_gather(hrow[b], [r16, md8])
                    plsc.store_scatter(orow[b], [r16, md8], wv * hv)
                return c2
            lax.fori_loop(0, CH // 16, emul, 0)

            for j in range(8):
                pltpu.async_copy(orow[b].at[pl.ds(j * LANES, LANES)],
                                 acc_sh.at[sdix[b].at[j]], ss[b], add=True)

            @pl.when(u < TPW // 2 - 1)
            def _():
                issue(b, t + 2)
        return carry
    lax.fori_loop(0, TPW // 2, step, 0)

    for b in range(2):
        pltpu.make_async_copy(rec_hbm.at[pl.ds(0, CH)], orow[b],
                              ss[b]).wait()
    plsc.subcore_barrier()

    @pl.when(cid == 0)
    def _():
        pltpu.sync_copy(acc_sh.at[pl.ds(row0, ROWS_PT)],
                        acca_hbm.at[pl.ds(row0, ROWS_PT)])

    @pl.when(cid == 1)
    def _():
        pltpu.sync_copy(acc_sh.at[pl.ds(row0, ROWS_PT)],
                        accb_hbm.at[pl.ds(row0, ROWS_PT)])


_sc_edge = functools.partial(
    pl.kernel,
    out_type=(jax.ShapeDtypeStruct((NP, 8), _F32),
              jax.ShapeDtypeStruct((NP, 8), _F32)),
    mesh=plsc.VectorSubcoreMesh(core_axis_name="c", subcore_axis_name="s",
                                num_cores=2, num_subcores=16),
    compiler_params=pltpu.CompilerParams(needs_layout_passes=False,
                                         use_tc_tiling_on_sc=False),
    scratch_types=(
        [pltpu.VMEM((8, LANES), jnp.int32)] * 6
        + [pltpu.VMEM((CH, 8), _F32)] * 4
        + [pltpu.VMEM((CH,), _F32)] * 2
        + [pltpu.VMEM((CH,), _F32)]
        + [pltpu.VMEM((16,), _F32)]
        + [pltpu.VMEM((ZROWS, 8), _F32)]
        + [pltpu.VMEM_SHARED((NP, 8), _F32)]
        + [pltpu.SemaphoreType.DMA] * 6
    ),
)(_sc_edge_body)


def _cols(acc):
    return jnp.moveaxis(acc.reshape(NR, LANES, 8), 2, 0)


def _assemble_rec(hst, cfeat):
    ones = jnp.ones((1, NR, LANES), _F32)
    parts = [hst[:cfeat], hst[cfeat:cfeat + 1], ones]
    pad = 8 - (cfeat + 2)
    if pad:
        parts.append(jnp.zeros((pad, NR, LANES), _F32))
    return jnp.moveaxis(jnp.concatenate(parts, 0), 0, 2).reshape(NP, 8)


def kernel(x, edge_index, W1, a_src1, a_dst1, b1, W2, a_src2, a_dst2, b2,
           W3, a_src3, a_dst3, b3):
    xp = jnp.pad(x[:, 0], (0, NP - N)).reshape(NR, LANES)
    srcv = edge_index[0].reshape(E // LANES, LANES)
    dstv = edge_index[1].reshape(E // LANES, LANES)

    d0 = _tc_call(_d0_body, 1, 3, [(HID + 2, NR, LANES), (NR, LANES)])
    hst1, a1 = d0(xp, W1, a_src1, a_dst1)

    acc1a, acc1b = _sc_edge(_assemble_rec(hst1, HID),
                            hst1[HID + 1].reshape(NP),
                            a1.reshape(NP)[:16], srcv, dstv)

    fin12 = _tc_call(_make_fin_body(HID, HID), 4, 4,
                     [(HID + 2, NR, LANES), (NR, LANES)])
    hst2, a2 = fin12(_cols(acc1a), _cols(acc1b), hst1, a1,
                     W2, a_src2, a_dst2, b1)

    acc2a, acc2b = _sc_edge(_assemble_rec(hst2, HID),
                            hst2[HID + 1].reshape(NP),
                            a2.reshape(NP)[:16], srcv, dstv)

    fin23 = _tc_call(_make_fin_body(HID, 1), 4, 4,
                     [(3, NR, LANES), (NR, LANES)])
    hst3, a3 = fin23(_cols(acc2a), _cols(acc2b), hst2, a2,
                     W3, a_src3, a_dst3, b2)

    acc3a, acc3b = _sc_edge(_assemble_rec(hst3, 1),
                            hst3[2].reshape(NP),
                            a3.reshape(NP)[:16], srcv, dstv)

    d3 = _tc_call(_d3_body, 4, 1, [(NR, LANES)])
    out = d3(_cols(acc3a), _cols(acc3b), hst3, a3, b3)

    return out[0].reshape(NP)[:N][:, None]


# speedup vs baseline: 150.1401x; 1.0060x over previous
import functools

import jax
import jax.numpy as jnp
from jax import lax
from jax.experimental import pallas as pl
from jax.experimental.pallas import tpu as pltpu
from jax.experimental.pallas import tpu_sc as plsc

N = 50000
HID = 6
E = 1600000
LANES = 128
NR = 391
NP = NR * LANES
NW = 32
CH = 1024
NCHUNK = E // CH
TPW = NCHUNK // NW
ROWS_PT = NP // 16
ZROWS = 782

_F32 = jnp.float32


def _node_mask():
    r = lax.broadcasted_iota(jnp.int32, (NR, LANES), 0)
    c = lax.broadcasted_iota(jnp.int32, (NR, LANES), 1)
    return r * LANES + c < N


def _leaky(v):
    return jnp.maximum(v, 0.2 * v)


def _d0_body(x_ref, w1_ref, asr_ref, adr_ref, hst_ref, acol_ref):
    x = x_ref[...]
    for c in range(HID):
        hst_ref[c] = x * w1_ref[0, c]
    cs = sum(w1_ref[0, k] * asr_ref[k] for k in range(HID))
    cd = sum(w1_ref[0, k] * adr_ref[k] for k in range(HID))
    asv = x * cs
    adv = x * cd
    hst_ref[HID] = asv
    hst_ref[HID + 1] = adv
    amax = jnp.max(jnp.where(_node_mask(), asv, -jnp.inf))
    acol_ref[...] = jnp.full((NR, LANES), amax, _F32)


def _make_fin_body(cin, cout):
    def body(aa_ref, ab_ref, hst_ref, acol_ref, w_ref, asr_ref, adr_ref,
             b_ref, ohst_ref, oacol_ref):
        asv = hst_ref[cin]
        adv = hst_ref[cin + 1]
        wself = jnp.exp(_leaky(asv + adv) - _leaky(adv + acol_ref[...]))
        den = aa_ref[cin + 1] + ab_ref[cin + 1] + wself
        inv = 1.0 / (den + 1e-16)
        xs = []
        for c in range(cin):
            num = aa_ref[c] + ab_ref[c] + wself * hst_ref[c]
            xs.append(jnp.maximum(num * inv + b_ref[c], 0.0))
        hn = []
        for c in range(cout):
            acc = xs[0] * w_ref[0, c]
            for k in range(1, cin):
                acc = acc + xs[k] * w_ref[k, c]
            hn.append(acc)
            ohst_ref[c] = acc
        asn = hn[0] * asr_ref[0]
        adn = hn[0] * adr_ref[0]
        for c in range(1, cout):
            asn = asn + hn[c] * asr_ref[c]
            adn = adn + hn[c] * adr_ref[c]
        ohst_ref[cout] = asn
        ohst_ref[cout + 1] = adn
        amax = jnp.max(jnp.where(_node_mask(), asn, -jnp.inf))
        oacol_ref[...] = jnp.full((NR, LANES), amax, _F32)
    return body


def _d3_body(aa_ref, ab_ref, hst_ref, acol_ref, b_ref, out_ref):
    asv = hst_ref[1]
    adv = hst_ref[2]
    wself = jnp.exp(_leaky(asv + adv) - _leaky(adv + acol_ref[...]))
    den = aa_ref[2] + ab_ref[2] + wself
    z = (aa_ref[0] + ab_ref[0] + wself * hst_ref[0]) / (den + 1e-16) + b_ref[0]
    mask = _node_mask()
    zmax = jnp.max(jnp.where(mask, z, -jnp.inf))
    ez = jnp.where(mask, jnp.exp(z - zmax), 0.0)
    out_ref[...] = ez / jnp.sum(ez)


_SMEM = pl.BlockSpec(memory_space=pltpu.MemorySpace.SMEM)
_VSPEC = pl.BlockSpec(memory_space=pltpu.MemorySpace.VMEM)


def _tc_call(body, n_in_vmem, n_in_smem, out_shapes):
    return pl.pallas_call(
        body,
        in_specs=[_VSPEC] * n_in_vmem + [_SMEM] * n_in_smem,
        out_specs=[_VSPEC] * len(out_shapes),
        out_shape=[jax.ShapeDtypeStruct(s, _F32) for s in out_shapes],
    )


def _sc_edge_body(rec_hbm, ad_hbm, av_hbm, srcv_hbm, dstv_hbm,
                  acca_hbm, accb_hbm,
                  srcidx0, srcidx1, dstidx0, dstidx1, sdix0, sdix1,
                  hrow0, hrow1, orow0, orow1, adbuf0, adbuf1,
                  wbuf, a16, zbuf, acc_sh, rec_sh, ad_sh,
                  sg0, sg1, sa0, sa1, ss0, ss1):
    srcidx = (srcidx0, srcidx1)
    dstidx = (dstidx0, dstidx1)
    sdix = (sdix0, sdix1)
    hrow = (hrow0, hrow1)
    orow = (orow0, orow1)
    adbuf = (adbuf0, adbuf1)
    sg = (sg0, sg1)
    sa = (sa0, sa1)
    ss = (ss0, ss1)

    cid = lax.axis_index("c")
    sid = lax.axis_index("s")
    wid = sid * 2 + cid

    pltpu.sync_copy(av_hbm, a16)
    av = a16[...]
    iota = lax.iota(jnp.int32, 16)
    dv8 = iota // 8
    md8 = iota % 8
    col6 = jnp.full((16,), HID, jnp.int32)
    zero16 = jnp.zeros((16,), _F32)

    def zb(j, carry):
        plsc.store_scatter(zbuf, [2 * j + dv8, md8], zero16)
        return carry
    lax.fori_loop(0, ZROWS * 8 // 16, zb, 0)

    row0 = sid * ROWS_PT
    pltpu.sync_copy(rec_hbm.at[pl.ds(row0, ROWS_PT)],
                    rec_sh.at[pl.ds(row0, ROWS_PT)])
    pltpu.sync_copy(ad_hbm.at[pl.ds(row0, ROWS_PT)],
                    ad_sh.at[pl.ds(row0, ROWS_PT)])
    for q in range(4):
        pltpu.sync_copy(zbuf, acc_sh.at[pl.ds(row0 + q * ZROWS, ZROWS)])
    plsc.subcore_barrier()

    def issue(b, t):
        g = wid + NW * t
        pltpu.sync_copy(srcv_hbm.at[pl.ds(g * 8, 8)], srcidx[b])
        pltpu.sync_copy(dstv_hbm.at[pl.ds(g * 8, 8)], dstidx[b])
        for j in range(8):
            pltpu.async_copy(rec_sh.at[srcidx[b].at[j]],
                             hrow[b].at[pl.ds(j * LANES, LANES)], sg[b])
            pltpu.async_copy(ad_sh.at[dstidx[b].at[j]],
                             adbuf[b].at[pl.ds(j * LANES, LANES)], sa[b])

    for b in range(2):
        issue(b, b)

    def step(u, carry):
        for b in range(2):
            t = 2 * u + b
            pltpu.make_async_copy(rec_hbm.at[pl.ds(0, CH)], hrow[b],
                                  sg[b]).wait()
            pltpu.make_async_copy(ad_hbm.at[pl.ds(0, CH)], adbuf[b],
                                  sa[b]).wait()

            @pl.when(u >= 1)
            def _():
                pltpu.make_async_copy(rec_hbm.at[pl.ds(0, CH)], orow[b],
                                      ss[b]).wait()

            def wcomp(k, c2):
                e16 = k * 16 + iota
                asv = plsc.load_gather(hrow[b], [e16, col6])
                adv = plsc.load_gather(adbuf[b], [e16])
                s = asv + adv
                e = jnp.maximum(s, 0.2 * s)
                tt = adv + av
                mp = jnp.maximum(tt, 0.2 * tt)
                plsc.store_scatter(wbuf, [e16], jnp.exp(e - mp))
                r7 = e16 >> 7
                c7 = e16 & 127
                iv = plsc.load_gather(dstidx[b], [r7, c7])
                plsc.store_scatter(sdix[b], [r7, c7], iv)
                return c2
            lax.fori_loop(0, CH // 16, wcomp, 0)

            def emul(k, c2):
                for p in range(16):
                    r16 = (k * 32 + 2 * p) + dv8
                    wv = plsc.load_gather(wbuf, [r16])
                    hv = plsc.load_gather(hrow[b], [r16, md8])
                    plsc.store_scatter(orow[b], [r16, md8], wv * hv)
                return c2
            lax.fori_loop(0, CH // 32, emul, 0)

            for j in range(8):
                pltpu.async_copy(orow[b].at[pl.ds(j * LANES, LANES)],
                                 acc_sh.at[sdix[b].at[j]], ss[b], add=True)

            @pl.when(u < TPW // 2 - 1)
            def _():
                issue(b, t + 2)
        return carry
    lax.fori_loop(0, TPW // 2, step, 0)

    for b in range(2):
        pltpu.make_async_copy(rec_hbm.at[pl.ds(0, CH)], orow[b],
                              ss[b]).wait()
    plsc.subcore_barrier()

    @pl.when(cid == 0)
    def _():
        pltpu.sync_copy(acc_sh.at[pl.ds(row0, ROWS_PT)],
                        acca_hbm.at[pl.ds(row0, ROWS_PT)])

    @pl.when(cid == 1)
    def _():
        pltpu.sync_copy(acc_sh.at[pl.ds(row0, ROWS_PT)],
                        accb_hbm.at[pl.ds(row0, ROWS_PT)])


_sc_edge = functools.partial(
    pl.kernel,
    out_type=(jax.ShapeDtypeStruct((NP, 8), _F32),
              jax.ShapeDtypeStruct((NP, 8), _F32)),
    mesh=plsc.VectorSubcoreMesh(core_axis_name="c", subcore_axis_name="s",
                                num_cores=2, num_subcores=16),
    compiler_params=pltpu.CompilerParams(needs_layout_passes=False,
                                         use_tc_tiling_on_sc=False),
    scratch_types=(
        [pltpu.VMEM((8, LANES), jnp.int32)] * 6
        + [pltpu.VMEM((CH, 8), _F32)] * 4
        + [pltpu.VMEM((CH,), _F32)] * 2
        + [pltpu.VMEM((CH,), _F32)]
        + [pltpu.VMEM((16,), _F32)]
        + [pltpu.VMEM((ZROWS, 8), _F32)]
        + [pltpu.VMEM_SHARED((NP, 8), _F32)]
        + [pltpu.VMEM_SHARED((NP, 8), _F32)]
        + [pltpu.VMEM_SHARED((NP,), _F32)]
        + [pltpu.SemaphoreType.DMA] * 6
    ),
)(_sc_edge_body)


def _cols(acc):
    return jnp.moveaxis(acc.reshape(NR, LANES, 8), 2, 0)


def _assemble_rec(hst, cfeat):
    ones = jnp.ones((1, NR, LANES), _F32)
    parts = [hst[:cfeat], hst[cfeat:cfeat + 1], ones]
    pad = 8 - (cfeat + 2)
    if pad:
        parts.append(jnp.zeros((pad, NR, LANES), _F32))
    return jnp.moveaxis(jnp.concatenate(parts, 0), 0, 2).reshape(NP, 8)


def kernel(x, edge_index, W1, a_src1, a_dst1, b1, W2, a_src2, a_dst2, b2,
           W3, a_src3, a_dst3, b3):
    xp = jnp.pad(x[:, 0], (0, NP - N)).reshape(NR, LANES)
    srcv = edge_index[0].reshape(E // LANES, LANES)
    dstv = edge_index[1].reshape(E // LANES, LANES)

    d0 = _tc_call(_d0_body, 1, 3, [(HID + 2, NR, LANES), (NR, LANES)])
    hst1, a1 = d0(xp, W1, a_src1, a_dst1)

    acc1a, acc1b = _sc_edge(_assemble_rec(hst1, HID),
                            hst1[HID + 1].reshape(NP),
                            a1.reshape(NP)[:16], srcv, dstv)

    fin12 = _tc_call(_make_fin_body(HID, HID), 4, 4,
                     [(HID + 2, NR, LANES), (NR, LANES)])
    hst2, a2 = fin12(_cols(acc1a), _cols(acc1b), hst1, a1,
                     W2, a_src2, a_dst2, b1)

    acc2a, acc2b = _sc_edge(_assemble_rec(hst2, HID),
                            hst2[HID + 1].reshape(NP),
                            a2.reshape(NP)[:16], srcv, dstv)

    fin23 = _tc_call(_make_fin_body(HID, 1), 4, 4,
                     [(3, NR, LANES), (NR, LANES)])
    hst3, a3 = fin23(_cols(acc2a), _cols(acc2b), hst2, a2,
                     W3, a_src3, a_dst3, b2)

    acc3a, acc3b = _sc_edge(_assemble_rec(hst3, 1),
                            hst3[2].reshape(NP),
                            a3.reshape(NP)[:16], srcv, dstv)

    d3 = _tc_call(_d3_body, 4, 1, [(NR, LANES)])
    out = d3(_cols(acc3a), _cols(acc3b), hst3, a3, b3)

    return out[0].reshape(NP)[:N][:, None]


# speedup vs baseline: 169.1129x; 1.1264x over previous
import functools

import jax
import jax.numpy as jnp
from jax import lax
from jax.experimental import pallas as pl
from jax.experimental.pallas import tpu as pltpu
from jax.experimental.pallas import tpu_sc as plsc

N = 50000
HID = 6
E = 1600000
LANES = 128
NR = 391
NP = NR * LANES
NW = 32
CH = 1024
NCHUNK = E // CH
TPW = NCHUNK // NW
ROWS_PT = NP // 16
ZROWS = 782

_F32 = jnp.float32


def _node_mask():
    r = lax.broadcasted_iota(jnp.int32, (NR, LANES), 0)
    c = lax.broadcasted_iota(jnp.int32, (NR, LANES), 1)
    return r * LANES + c < N


def _leaky(v):
    return jnp.maximum(v, 0.2 * v)


def _d0_body(x_ref, w1_ref, asr_ref, adr_ref, hst_ref, acol_ref):
    x = x_ref[...]
    for c in range(HID):
        hst_ref[c] = x * w1_ref[0, c]
    cs = sum(w1_ref[0, k] * asr_ref[k] for k in range(HID))
    cd = sum(w1_ref[0, k] * adr_ref[k] for k in range(HID))
    asv = x * cs
    adv = x * cd
    hst_ref[HID] = asv
    hst_ref[HID + 1] = adv
    amax = jnp.max(jnp.where(_node_mask(), asv, -jnp.inf))
    acol_ref[...] = jnp.full((NR, LANES), amax, _F32)


def _make_fin_body(cin, cout):
    def body(aa_ref, ab_ref, hst_ref, acol_ref, w_ref, asr_ref, adr_ref,
             b_ref, ohst_ref, oacol_ref):
        asv = hst_ref[cin]
        adv = hst_ref[cin + 1]
        wself = jnp.exp(_leaky(asv + adv) - _leaky(adv + acol_ref[...]))
        den = aa_ref[cin + 1] + ab_ref[cin + 1] + wself
        inv = 1.0 / (den + 1e-16)
        xs = []
        for c in range(cin):
            num = aa_ref[c] + ab_ref[c] + wself * hst_ref[c]
            xs.append(jnp.maximum(num * inv + b_ref[c], 0.0))
        hn = []
        for c in range(cout):
            acc = xs[0] * w_ref[0, c]
            for k in range(1, cin):
                acc = acc + xs[k] * w_ref[k, c]
            hn.append(acc)
            ohst_ref[c] = acc
        asn = hn[0] * asr_ref[0]
        adn = hn[0] * adr_ref[0]
        for c in range(1, cout):
            asn = asn + hn[c] * asr_ref[c]
            adn = adn + hn[c] * adr_ref[c]
        ohst_ref[cout] = asn
        ohst_ref[cout + 1] = adn
        amax = jnp.max(jnp.where(_node_mask(), asn, -jnp.inf))
        oacol_ref[...] = jnp.full((NR, LANES), amax, _F32)
    return body


def _d3_body(aa_ref, ab_ref, hst_ref, acol_ref, b_ref, out_ref):
    asv = hst_ref[1]
    adv = hst_ref[2]
    wself = jnp.exp(_leaky(asv + adv) - _leaky(adv + acol_ref[...]))
    den = aa_ref[2] + ab_ref[2] + wself
    z = (aa_ref[0] + ab_ref[0] + wself * hst_ref[0]) / (den + 1e-16) + b_ref[0]
    mask = _node_mask()
    zmax = jnp.max(jnp.where(mask, z, -jnp.inf))
    ez = jnp.where(mask, jnp.exp(z - zmax), 0.0)
    out_ref[...] = ez / jnp.sum(ez)


_SMEM = pl.BlockSpec(memory_space=pltpu.MemorySpace.SMEM)
_VSPEC = pl.BlockSpec(memory_space=pltpu.MemorySpace.VMEM)


def _tc_call(body, n_in_vmem, n_in_smem, out_shapes):
    return pl.pallas_call(
        body,
        in_specs=[_VSPEC] * n_in_vmem + [_SMEM] * n_in_smem,
        out_specs=[_VSPEC] * len(out_shapes),
        out_shape=[jax.ShapeDtypeStruct(s, _F32) for s in out_shapes],
    )


def _sc_edge_body(rec_hbm, ad_hbm, av_hbm, srcv_hbm, dstv_hbm,
                  acca_hbm, accb_hbm,
                  srcidx0, srcidx1, dstidx0, dstidx1, sdix0, sdix1,
                  hrow0, hrow1, orow0, orow1, adbuf0, adbuf1,
                  a16, zbuf, acc_sh, rec_sh, ad_sh,
                  sg0, sg1, sa0, sa1, ss0, ss1):
    srcidx = (srcidx0, srcidx1)
    dstidx = (dstidx0, dstidx1)
    sdix = (sdix0, sdix1)
    hrow = (hrow0, hrow1)
    orow = (orow0, orow1)
    adbuf = (adbuf0, adbuf1)
    sg = (sg0, sg1)
    sa = (sa0, sa1)
    ss = (ss0, ss1)

    cid = lax.axis_index("c")
    sid = lax.axis_index("s")
    wid = sid * 2 + cid

    pltpu.sync_copy(av_hbm, a16)
    av = a16[...]
    iota = lax.iota(jnp.int32, 16)
    dv8 = iota // 8
    md8 = iota % 8
    col6 = jnp.full((16,), HID, jnp.int32)
    col7 = jnp.full((16,), 7, jnp.int32)
    zero16 = jnp.zeros((16,), _F32)

    def zb(j, carry):
        plsc.store_scatter(zbuf, [2 * j + dv8, md8], zero16)
        return carry
    lax.fori_loop(0, ZROWS * 8 // 16, zb, 0)

    row0 = sid * ROWS_PT
    pltpu.sync_copy(rec_hbm.at[pl.ds(row0, ROWS_PT)],
                    rec_sh.at[pl.ds(row0, ROWS_PT)])
    pltpu.sync_copy(ad_hbm.at[pl.ds(row0, ROWS_PT)],
                    ad_sh.at[pl.ds(row0, ROWS_PT)])
    for q in range(4):
        pltpu.sync_copy(zbuf, acc_sh.at[pl.ds(row0 + q * ZROWS, ZROWS)])
    plsc.subcore_barrier()

    def issue(b, t):
        g = wid + NW * t
        pltpu.sync_copy(srcv_hbm.at[pl.ds(g * 8, 8)], srcidx[b])
        pltpu.sync_copy(dstv_hbm.at[pl.ds(g * 8, 8)], dstidx[b])
        for j in range(8):
            pltpu.async_copy(rec_sh.at[srcidx[b].at[j]],
                             hrow[b].at[pl.ds(j * LANES, LANES)], sg[b])
            pltpu.async_copy(ad_sh.at[dstidx[b].at[j]],
                             adbuf[b].at[pl.ds(j * LANES, LANES)], sa[b])

    for b in range(2):
        issue(b, b)

    def step(u, carry):
        for b in range(2):
            t = 2 * u + b
            pltpu.make_async_copy(rec_hbm.at[pl.ds(0, CH)], hrow[b],
                                  sg[b]).wait()
            pltpu.make_async_copy(ad_hbm.at[pl.ds(0, CH)], adbuf[b],
                                  sa[b]).wait()

            @pl.when(u >= 1)
            def _():
                pltpu.make_async_copy(rec_hbm.at[pl.ds(0, CH)], orow[b],
                                      ss[b]).wait()

            def fused(k, c2):
                e16 = k * 16 + iota
                asv = plsc.load_gather(hrow[b], [e16, col6])
                adv = plsc.load_gather(adbuf[b], [e16])
                s = asv + adv
                e = jnp.maximum(s, 0.2 * s)
                tt = adv + av
                mp = jnp.maximum(tt, 0.2 * tt)
                w = jnp.exp(e - mp)
                r7 = e16 >> 7
                c7 = e16 & 127
                iv = plsc.load_gather(dstidx[b], [r7, c7])
                plsc.store_scatter(sdix[b], [r7, c7], iv)
                for f in range(HID):
                    cf = jnp.full((16,), f, jnp.int32)
                    hf = plsc.load_gather(hrow[b], [e16, cf])
                    plsc.store_scatter(orow[b], [e16, cf], w * hf)
                plsc.store_scatter(orow[b], [e16, col7], w)
                return c2
            lax.fori_loop(0, CH // 16, fused, 0)

            for j in range(8):
                pltpu.async_copy(orow[b].at[pl.ds(j * LANES, LANES)],
                                 acc_sh.at[sdix[b].at[j]], ss[b], add=True)

            @pl.when(u < TPW // 2 - 1)
            def _():
                issue(b, t + 2)
        return carry
    lax.fori_loop(0, TPW // 2, step, 0)

    for b in range(2):
        pltpu.make_async_copy(rec_hbm.at[pl.ds(0, CH)], orow[b],
                              ss[b]).wait()
    plsc.subcore_barrier()

    @pl.when(cid == 0)
    def _():
        pltpu.sync_copy(acc_sh.at[pl.ds(row0, ROWS_PT)],
                        acca_hbm.at[pl.ds(row0, ROWS_PT)])

    @pl.when(cid == 1)
    def _():
        pltpu.sync_copy(acc_sh.at[pl.ds(row0, ROWS_PT)],
                        accb_hbm.at[pl.ds(row0, ROWS_PT)])


_sc_edge = functools.partial(
    pl.kernel,
    out_type=(jax.ShapeDtypeStruct((NP, 8), _F32),
              jax.ShapeDtypeStruct((NP, 8), _F32)),
    mesh=plsc.VectorSubcoreMesh(core_axis_name="c", subcore_axis_name="s",
                                num_cores=2, num_subcores=16),
    compiler_params=pltpu.CompilerParams(needs_layout_passes=False,
                                         use_tc_tiling_on_sc=False),
    scratch_types=(
        [pltpu.VMEM((8, LANES), jnp.int32)] * 6
        + [pltpu.VMEM((CH, 8), _F32)] * 4
        + [pltpu.VMEM((CH,), _F32)] * 2
        + [pltpu.VMEM((16,), _F32)]
        + [pltpu.VMEM((ZROWS, 8), _F32)]
        + [pltpu.VMEM_SHARED((NP, 8), _F32)]
        + [pltpu.VMEM_SHARED((NP, 8), _F32)]
        + [pltpu.VMEM_SHARED((NP,), _F32)]
        + [pltpu.SemaphoreType.DMA] * 6
    ),
)(_sc_edge_body)


def _cols(acc):
    return jnp.moveaxis(acc.reshape(NR, LANES, 8), 2, 0)


def _assemble_rec(hst, cfeat):
    ones = jnp.ones((1, NR, LANES), _F32)
    parts = [hst[:cfeat], hst[cfeat:cfeat + 1], ones]
    pad = 8 - (cfeat + 2)
    if pad:
        parts.append(jnp.zeros((pad, NR, LANES), _F32))
    return jnp.moveaxis(jnp.concatenate(parts, 0), 0, 2).reshape(NP, 8)


def kernel(x, edge_index, W1, a_src1, a_dst1, b1, W2, a_src2, a_dst2, b2,
           W3, a_src3, a_dst3, b3):
    xp = jnp.pad(x[:, 0], (0, NP - N)).reshape(NR, LANES)
    srcv = edge_index[0].reshape(E // LANES, LANES)
    dstv = edge_index[1].reshape(E // LANES, LANES)

    d0 = _tc_call(_d0_body, 1, 3, [(HID + 2, NR, LANES), (NR, LANES)])
    hst1, a1 = d0(xp, W1, a_src1, a_dst1)

    acc1a, acc1b = _sc_edge(_assemble_rec(hst1, HID),
                            hst1[HID + 1].reshape(NP),
                            a1.reshape(NP)[:16], srcv, dstv)

    fin12 = _tc_call(_make_fin_body(HID, HID), 4, 4,
                     [(HID + 2, NR, LANES), (NR, LANES)])
    hst2, a2 = fin12(_cols(acc1a), _cols(acc1b), hst1, a1,
                     W2, a_src2, a_dst2, b1)

    acc2a, acc2b = _sc_edge(_assemble_rec(hst2, HID),
                            hst2[HID + 1].reshape(NP),
                            a2.reshape(NP)[:16], srcv, dstv)

    fin23 = _tc_call(_make_fin_body(HID, 1), 4, 4,
                     [(3, NR, LANES), (NR, LANES)])
    hst3, a3 = fin23(_cols(acc2a), _cols(acc2b), hst2, a2,
                     W3, a_src3, a_dst3, b2)

    acc3a, acc3b = _sc_edge(_assemble_rec(hst3, 1),
                            hst3[2].reshape(NP),
                            a3.reshape(NP)[:16], srcv, dstv)

    d3 = _tc_call(_d3_body, 4, 1, [(NR, LANES)])
    out = d3(_cols(acc3a), _cols(acc3b), hst3, a3, b3)

    return out[0].reshape(NP)[:N][:, None]


# speedup vs baseline: 172.2014x; 1.0183x over previous
import functools

import jax
import jax.numpy as jnp
from jax import lax
from jax.experimental import pallas as pl
from jax.experimental.pallas import tpu as pltpu
from jax.experimental.pallas import tpu_sc as plsc

N = 50000
HID = 6
E = 1600000
LANES = 128
NR = 391
NP = NR * LANES
NW = 32
CH = 1024
EPAD = 1638400
NCHUNK = EPAD // CH
TPW = NCHUNK // NW
ROWS_PT = NP // 16
ZROWS = 782

_F32 = jnp.float32


def _node_mask():
    r = lax.broadcasted_iota(jnp.int32, (NR, LANES), 0)
    c = lax.broadcasted_iota(jnp.int32, (NR, LANES), 1)
    return r * LANES + c < N


def _leaky(v):
    return jnp.maximum(v, 0.2 * v)


def _d0_body(x_ref, w1_ref, asr_ref, adr_ref, hst_ref, acol_ref):
    x = x_ref[...]
    for c in range(HID):
        hst_ref[c] = x * w1_ref[0, c]
    cs = sum(w1_ref[0, k] * asr_ref[k] for k in range(HID))
    cd = sum(w1_ref[0, k] * adr_ref[k] for k in range(HID))
    asv = x * cs
    adv = x * cd
    hst_ref[HID] = asv
    hst_ref[HID + 1] = adv
    amax = jnp.max(jnp.where(_node_mask(), asv, -jnp.inf))
    acol_ref[...] = jnp.full((NR, LANES), amax, _F32)


def _make_fin_body(cin, cout):
    def body(aa_ref, ab_ref, hst_ref, acol_ref, w_ref, asr_ref, adr_ref,
             b_ref, ohst_ref, oacol_ref):
        asv = hst_ref[cin]
        adv = hst_ref[cin + 1]
        wself = jnp.exp(_leaky(asv + adv) - _leaky(adv + acol_ref[...]))
        den = aa_ref[cin + 1] + ab_ref[cin + 1] + wself
        inv = 1.0 / (den + 1e-16)
        xs = []
        for c in range(cin):
            num = aa_ref[c] + ab_ref[c] + wself * hst_ref[c]
            xs.append(jnp.maximum(num * inv + b_ref[c], 0.0))
        hn = []
        for c in range(cout):
            acc = xs[0] * w_ref[0, c]
            for k in range(1, cin):
                acc = acc + xs[k] * w_ref[k, c]
            hn.append(acc)
            ohst_ref[c] = acc
        asn = hn[0] * asr_ref[0]
        adn = hn[0] * adr_ref[0]
        for c in range(1, cout):
            asn = asn + hn[c] * asr_ref[c]
            adn = adn + hn[c] * adr_ref[c]
        ohst_ref[cout] = asn
        ohst_ref[cout + 1] = adn
        amax = jnp.max(jnp.where(_node_mask(), asn, -jnp.inf))
        oacol_ref[...] = jnp.full((NR, LANES), amax, _F32)
    return body


def _d3_body(aa_ref, ab_ref, hst_ref, acol_ref, b_ref, out_ref):
    asv = hst_ref[1]
    adv = hst_ref[2]
    wself = jnp.exp(_leaky(asv + adv) - _leaky(adv + acol_ref[...]))
    den = aa_ref[2] + ab_ref[2] + wself
    z = (aa_ref[0] + ab_ref[0] + wself * hst_ref[0]) / (den + 1e-16) + b_ref[0]
    mask = _node_mask()
    zmax = jnp.max(jnp.where(mask, z, -jnp.inf))
    ez = jnp.where(mask, jnp.exp(z - zmax), 0.0)
    out_ref[...] = ez / jnp.sum(ez)


_SMEM = pl.BlockSpec(memory_space=pltpu.MemorySpace.SMEM)
_VSPEC = pl.BlockSpec(memory_space=pltpu.MemorySpace.VMEM)


def _tc_call(body, n_in_vmem, n_in_smem, out_shapes):
    return pl.pallas_call(
        body,
        in_specs=[_VSPEC] * n_in_vmem + [_SMEM] * n_in_smem,
        out_specs=[_VSPEC] * len(out_shapes),
        out_shape=[jax.ShapeDtypeStruct(s, _F32) for s in out_shapes],
    )


def _sc_edge_body(rec_hbm, ad_hbm, av_hbm, srcv_hbm, dstv_hbm,
                  acca_hbm, accb_hbm,
                  srcidx0, srcidx1, dstidx0, dstidx1, sdix0, sdix1,
                  hrow0, hrow1, orow0, orow1, adbuf0, adbuf1,
                  a16, zbuf, acc_sh, rec_sh, ad_sh,
                  sg0, sg1, sa0, sa1, ss0, ss1):
    srcidx = (srcidx0, srcidx1)
    dstidx = (dstidx0, dstidx1)
    sdix = (sdix0, sdix1)
    hrow = (hrow0, hrow1)
    orow = (orow0, orow1)
    adbuf = (adbuf0, adbuf1)
    sg = (sg0, sg1)
    sa = (sa0, sa1)
    ss = (ss0, ss1)

    cid = lax.axis_index("c")
    sid = lax.axis_index("s")
    wid = sid * 2 + cid

    pltpu.sync_copy(av_hbm, a16)
    av = a16[...]
    iota = lax.iota(jnp.int32, 16)
    dv8 = iota // 8
    md8 = iota % 8
    col6 = jnp.full((16,), HID, jnp.int32)
    col7 = jnp.full((16,), 7, jnp.int32)
    zero16 = jnp.zeros((16,), _F32)

    def zb(j, carry):
        plsc.store_scatter(zbuf, [2 * j + dv8, md8], zero16)
        return carry
    lax.fori_loop(0, ZROWS * 8 // 16, zb, 0)

    row0 = sid * ROWS_PT
    pltpu.sync_copy(rec_hbm.at[pl.ds(row0, ROWS_PT)],
                    rec_sh.at[pl.ds(row0, ROWS_PT)])
    pltpu.sync_copy(ad_hbm.at[pl.ds(row0, ROWS_PT)],
                    ad_sh.at[pl.ds(row0, ROWS_PT)])
    for q in range(4):
        pltpu.sync_copy(zbuf, acc_sh.at[pl.ds(row0 + q * ZROWS, ZROWS)])
    plsc.subcore_barrier()

    def issue(b, t):
        g = wid * TPW + t
        pltpu.sync_copy(srcv_hbm.at[pl.ds(g * 8, 8)], srcidx[b])
        pltpu.sync_copy(dstv_hbm.at[pl.ds(g * 8, 8)], dstidx[b])
        for j in range(8):
            pltpu.async_copy(rec_sh.at[srcidx[b].at[j]],
                             hrow[b].at[pl.ds(j * LANES, LANES)], sg[b])
            pltpu.async_copy(ad_sh.at[dstidx[b].at[j]],
                             adbuf[b].at[pl.ds(j * LANES, LANES)], sa[b])

    for b in range(2):
        issue(b, b)

    def step(u, carry):
        for b in range(2):
            t = 2 * u + b
            pltpu.make_async_copy(rec_hbm.at[pl.ds(0, CH)], hrow[b],
                                  sg[b]).wait()
            pltpu.make_async_copy(ad_hbm.at[pl.ds(0, CH)], adbuf[b],
                                  sa[b]).wait()

            @pl.when(u >= 1)
            def _():
                pltpu.make_async_copy(rec_hbm.at[pl.ds(0, CH)], orow[b],
                                      ss[b]).wait()

            def fused(k, c2):
                e16 = k * 16 + iota
                asv = plsc.load_gather(hrow[b], [e16, col6])
                adv = plsc.load_gather(adbuf[b], [e16])
                s = asv + adv
                e = jnp.maximum(s, 0.2 * s)
                tt = adv + av
                mp = jnp.maximum(tt, 0.2 * tt)
                w = jnp.exp(e - mp)
                r7 = e16 >> 7
                c7 = e16 & 127
                iv = plsc.load_gather(dstidx[b], [r7, c7])
                plsc.store_scatter(sdix[b], [r7, c7], iv)
                for f in range(HID):
                    cf = jnp.full((16,), f, jnp.int32)
                    hf = plsc.load_gather(hrow[b], [e16, cf])
                    plsc.store_scatter(orow[b], [e16, cf], w * hf)
                plsc.store_scatter(orow[b], [e16, col7], w)
                return c2
            lax.fori_loop(0, CH // 16, fused, 0)

            for j in range(8):
                pltpu.async_copy(orow[b].at[pl.ds(j * LANES, LANES)],
                                 acc_sh.at[sdix[b].at[j]], ss[b], add=True)

            @pl.when(u < TPW // 2 - 1)
            def _():
                issue(b, t + 2)
        return carry
    lax.fori_loop(0, TPW // 2, step, 0)

    for b in range(2):
        pltpu.make_async_copy(rec_hbm.at[pl.ds(0, CH)], orow[b],
                              ss[b]).wait()
    plsc.subcore_barrier()

    @pl.when(cid == 0)
    def _():
        pltpu.sync_copy(acc_sh.at[pl.ds(row0, ROWS_PT)],
                        acca_hbm.at[pl.ds(row0, ROWS_PT)])

    @pl.when(cid == 1)
    def _():
        pltpu.sync_copy(acc_sh.at[pl.ds(row0, ROWS_PT)],
                        accb_hbm.at[pl.ds(row0, ROWS_PT)])


_sc_edge = functools.partial(
    pl.kernel,
    out_type=(jax.ShapeDtypeStruct((NP, 8), _F32),
              jax.ShapeDtypeStruct((NP, 8), _F32)),
    mesh=plsc.VectorSubcoreMesh(core_axis_name="c", subcore_axis_name="s",
                                num_cores=2, num_subcores=16),
    compiler_params=pltpu.CompilerParams(needs_layout_passes=False,
                                         use_tc_tiling_on_sc=False),
    scratch_types=(
        [pltpu.VMEM((8, LANES), jnp.int32)] * 6
        + [pltpu.VMEM((CH, 8), _F32)] * 4
        + [pltpu.VMEM((CH,), _F32)] * 2
        + [pltpu.VMEM((16,), _F32)]
        + [pltpu.VMEM((ZROWS, 8), _F32)]
        + [pltpu.VMEM_SHARED((NP, 8), _F32)]
        + [pltpu.VMEM_SHARED((NP, 8), _F32)]
        + [pltpu.VMEM_SHARED((NP,), _F32)]
        + [pltpu.SemaphoreType.DMA] * 6
    ),
)(_sc_edge_body)


def _cols(acc):
    return jnp.moveaxis(acc.reshape(NR, LANES, 8), 2, 0)


def _assemble_rec(hst, cfeat):
    ones = jnp.ones((1, NR, LANES), _F32)
    parts = [hst[:cfeat], hst[cfeat:cfeat + 1], ones]
    pad = 8 - (cfeat + 2)
    if pad:
        parts.append(jnp.zeros((pad, NR, LANES), _F32))
    return jnp.moveaxis(jnp.concatenate(parts, 0), 0, 2).reshape(NP, 8)


def kernel(x, edge_index, W1, a_src1, a_dst1, b1, W2, a_src2, a_dst2, b2,
           W3, a_src3, a_dst3, b3):
    xp = jnp.pad(x[:, 0], (0, NP - N)).reshape(NR, LANES)
    epad = jnp.full((2, EPAD - E), NP - 1, jnp.int32)
    ei = jnp.concatenate([edge_index, epad], axis=1)
    srcv = ei[0].reshape(EPAD // LANES, LANES)
    dstv = ei[1].reshape(EPAD // LANES, LANES)

    d0 = _tc_call(_d0_body, 1, 3, [(HID + 2, NR, LANES), (NR, LANES)])
    hst1, a1 = d0(xp, W1, a_src1, a_dst1)

    acc1a, acc1b = _sc_edge(_assemble_rec(hst1, HID),
                            hst1[HID + 1].reshape(NP),
                            a1.reshape(NP)[:16], srcv, dstv)

    fin12 = _tc_call(_make_fin_body(HID, HID), 4, 4,
                     [(HID + 2, NR, LANES), (NR, LANES)])
    hst2, a2 = fin12(_cols(acc1a), _cols(acc1b), hst1, a1,
                     W2, a_src2, a_dst2, b1)

    acc2a, acc2b = _sc_edge(_assemble_rec(hst2, HID),
                            hst2[HID + 1].reshape(NP),
                            a2.reshape(NP)[:16], srcv, dstv)

    fin23 = _tc_call(_make_fin_body(HID, 1), 4, 4,
                     [(3, NR, LANES), (NR, LANES)])
    hst3, a3 = fin23(_cols(acc2a), _cols(acc2b), hst2, a2,
                     W3, a_src3, a_dst3, b2)

    acc3a, acc3b = _sc_edge(_assemble_rec(hst3, 1),
                            hst3[2].reshape(NP),
                            a3.reshape(NP)[:16], srcv, dstv)

    d3 = _tc_call(_d3_body, 4, 1, [(NR, LANES)])
    out = d3(_cols(acc3a), _cols(acc3b), hst3, a3, b3)

    return out[0].reshape(NP)[:N][:, None]


# speedup vs baseline: 184.9362x; 1.0740x over previous
import functools

import jax
import jax.numpy as jnp
from jax import lax
from jax.experimental import pallas as pl
from jax.experimental.pallas import tpu as pltpu
from jax.experimental.pallas import tpu_sc as plsc

N = 50000
HID = 6
E = 1600000
LANES = 128
NR = 391
NP = NR * LANES
NW = 32
CH = 1024
EPAD = 1638400
NCHUNK = EPAD // CH
TPW = NCHUNK // NW
ROWS_PT = NP // 16
ZROWS = 782

_F32 = jnp.float32


def _node_mask():
    r = lax.broadcasted_iota(jnp.int32, (NR, LANES), 0)
    c = lax.broadcasted_iota(jnp.int32, (NR, LANES), 1)
    return r * LANES + c < N


def _leaky(v):
    return jnp.maximum(v, 0.2 * v)


def _d0_body(x_ref, w1_ref, asr_ref, adr_ref, hst_ref, acol_ref):
    x = x_ref[...]
    for c in range(HID):
        hst_ref[c] = x * w1_ref[0, c]
    cs = sum(w1_ref[0, k] * asr_ref[k] for k in range(HID))
    cd = sum(w1_ref[0, k] * adr_ref[k] for k in range(HID))
    asv = x * cs
    adv = x * cd
    hst_ref[HID] = asv
    hst_ref[HID + 1] = adv
    amax = jnp.max(jnp.where(_node_mask(), asv, -jnp.inf))
    acol_ref[...] = jnp.full((NR, LANES), amax, _F32)


def _make_fin_body(cin, cout):
    def body(aa_ref, ab_ref, hst_ref, acol_ref, w_ref, asr_ref, adr_ref,
             b_ref, ohst_ref, oacol_ref):
        asv = hst_ref[cin]
        adv = hst_ref[cin + 1]
        wself = jnp.exp(_leaky(asv + adv) - _leaky(adv + acol_ref[...]))
        den = aa_ref[cin + 1] + ab_ref[cin + 1] + wself
        inv = 1.0 / (den + 1e-16)
        xs = []
        for c in range(cin):
            num = aa_ref[c] + ab_ref[c] + wself * hst_ref[c]
            xs.append(jnp.maximum(num * inv + b_ref[c], 0.0))
        hn = []
        for c in range(cout):
            acc = xs[0] * w_ref[0, c]
            for k in range(1, cin):
                acc = acc + xs[k] * w_ref[k, c]
            hn.append(acc)
            ohst_ref[c] = acc
        asn = hn[0] * asr_ref[0]
        adn = hn[0] * adr_ref[0]
        for c in range(1, cout):
            asn = asn + hn[c] * asr_ref[c]
            adn = adn + hn[c] * adr_ref[c]
        ohst_ref[cout] = asn
        ohst_ref[cout + 1] = adn
        amax = jnp.max(jnp.where(_node_mask(), asn, -jnp.inf))
        oacol_ref[...] = jnp.full((NR, LANES), amax, _F32)
    return body


def _d3_body(aa_ref, ab_ref, hst_ref, acol_ref, b_ref, out_ref):
    asv = hst_ref[1]
    adv = hst_ref[2]
    wself = jnp.exp(_leaky(asv + adv) - _leaky(adv + acol_ref[...]))
    den = aa_ref[2] + ab_ref[2] + wself
    z = (aa_ref[0] + ab_ref[0] + wself * hst_ref[0]) / (den + 1e-16) + b_ref[0]
    mask = _node_mask()
    zmax = jnp.max(jnp.where(mask, z, -jnp.inf))
    ez = jnp.where(mask, jnp.exp(z - zmax), 0.0)
    out_ref[...] = ez / jnp.sum(ez)


_SMEM = pl.BlockSpec(memory_space=pltpu.MemorySpace.SMEM)
_VSPEC = pl.BlockSpec(memory_space=pltpu.MemorySpace.VMEM)


def _tc_call(body, n_in_vmem, n_in_smem, out_shapes):
    return pl.pallas_call(
        body,
        in_specs=[_VSPEC] * n_in_vmem + [_SMEM] * n_in_smem,
        out_specs=[_VSPEC] * len(out_shapes),
        out_shape=[jax.ShapeDtypeStruct(s, _F32) for s in out_shapes],
    )


def _sc_edge_body(rec_hbm, ad_hbm, av_hbm, srcv_hbm, dstv_hbm,
                  acca_hbm, accb_hbm,
                  srcidx0, srcidx1, dstidx0, dstidx1, sdix0, sdix1,
                  hrow0, hrow1, orow0, orow1, adbuf0, adbuf1,
                  a16, zbuf, acc_sh, rec_sh, ad_sh,
                  sg0, sg1, sa0, sa1, ss0, ss1, si0, si1):
    srcidx = (srcidx0, srcidx1)
    dstidx = (dstidx0, dstidx1)
    sdix = (sdix0, sdix1)
    hrow = (hrow0, hrow1)
    orow = (orow0, orow1)
    adbuf = (adbuf0, adbuf1)
    sg = (sg0, sg1)
    sa = (sa0, sa1)
    ss = (ss0, ss1)
    si = (si0, si1)

    cid = lax.axis_index("c")
    sid = lax.axis_index("s")
    wid = sid * 2 + cid

    pltpu.sync_copy(av_hbm, a16)
    av = a16[...]
    iota = lax.iota(jnp.int32, 16)
    dv8 = iota // 8
    md8 = iota % 8
    col6 = jnp.full((16,), HID, jnp.int32)
    col7 = jnp.full((16,), 7, jnp.int32)
    zero16 = jnp.zeros((16,), _F32)

    def zb(j, carry):
        plsc.store_scatter(zbuf, [2 * j + dv8, md8], zero16)
        return carry
    lax.fori_loop(0, ZROWS * 8 // 16, zb, 0)

    row0 = sid * ROWS_PT
    pltpu.sync_copy(rec_hbm.at[pl.ds(row0, ROWS_PT)],
                    rec_sh.at[pl.ds(row0, ROWS_PT)])
    pltpu.sync_copy(ad_hbm.at[pl.ds(row0, ROWS_PT)],
                    ad_sh.at[pl.ds(row0, ROWS_PT)])
    for q in range(4):
        pltpu.sync_copy(zbuf, acc_sh.at[pl.ds(row0 + q * ZROWS, ZROWS)])
    plsc.subcore_barrier()

    def issue_idx(b, t):
        g = wid * TPW + t
        pltpu.async_copy(srcv_hbm.at[pl.ds(g * 8, 8)], srcidx[b], si[b])
        pltpu.async_copy(dstv_hbm.at[pl.ds(g * 8, 8)], dstidx[b], si[b])

    def fire(b):
        pltpu.make_async_copy(srcv_hbm.at[pl.ds(0, 8)], srcidx[b],
                              si[b]).wait()
        pltpu.make_async_copy(srcv_hbm.at[pl.ds(0, 8)], dstidx[b],
                              si[b]).wait()
        for j in range(8):
            pltpu.async_copy(rec_sh.at[srcidx[b].at[j]],
                             hrow[b].at[pl.ds(j * LANES, LANES)], sg[b])
            pltpu.async_copy(ad_sh.at[dstidx[b].at[j]],
                             adbuf[b].at[pl.ds(j * LANES, LANES)], sa[b])

    issue_idx(0, 0)
    issue_idx(1, 1)
    fire(0)

    def step(u, carry):
        for b in range(2):
            t = 2 * u + b
            if b == 0:
                fire(1)
            else:
                @pl.when(u < TPW // 2 - 1)
                def _():
                    fire(0)
            pltpu.make_async_copy(rec_hbm.at[pl.ds(0, CH)], hrow[b],
                                  sg[b]).wait()
            pltpu.make_async_copy(ad_hbm.at[pl.ds(0, CH)], adbuf[b],
                                  sa[b]).wait()

            @pl.when(u >= 1)
            def _():
                pltpu.make_async_copy(rec_hbm.at[pl.ds(0, CH)], orow[b],
                                      ss[b]).wait()

            def fused(k, c2):
                e16 = k * 16 + iota
                asv = plsc.load_gather(hrow[b], [e16, col6])
                adv = plsc.load_gather(adbuf[b], [e16])
                s = asv + adv
                e = jnp.maximum(s, 0.2 * s)
                tt = adv + av
                mp = jnp.maximum(tt, 0.2 * tt)
                w = jnp.exp(e - mp)
                r7 = e16 >> 7
                c7 = e16 & 127
                iv = plsc.load_gather(dstidx[b], [r7, c7])
                plsc.store_scatter(sdix[b], [r7, c7], iv)
                for f in range(HID):
                    cf = jnp.full((16,), f, jnp.int32)
                    hf = plsc.load_gather(hrow[b], [e16, cf])
                    plsc.store_scatter(orow[b], [e16, cf], w * hf)
                plsc.store_scatter(orow[b], [e16, col7], w)
                return c2
            lax.fori_loop(0, CH // 16, fused, 0)

            for j in range(8):
                pltpu.async_copy(orow[b].at[pl.ds(j * LANES, LANES)],
                                 acc_sh.at[sdix[b].at[j]], ss[b], add=True)

            @pl.when(u < TPW // 2 - 1)
            def _():
                issue_idx(b, t + 2)
        return carry
    lax.fori_loop(0, TPW // 2, step, 0)

    for b in range(2):
        pltpu.make_async_copy(rec_hbm.at[pl.ds(0, CH)], orow[b],
                              ss[b]).wait()
    plsc.subcore_barrier()

    @pl.when(cid == 0)
    def _():
        pltpu.sync_copy(acc_sh.at[pl.ds(row0, ROWS_PT)],
                        acca_hbm.at[pl.ds(row0, ROWS_PT)])

    @pl.when(cid == 1)
    def _():
        pltpu.sync_copy(acc_sh.at[pl.ds(row0, ROWS_PT)],
                        accb_hbm.at[pl.ds(row0, ROWS_PT)])


_sc_edge = functools.partial(
    pl.kernel,
    out_type=(jax.ShapeDtypeStruct((NP, 8), _F32),
              jax.ShapeDtypeStruct((NP, 8), _F32)),
    mesh=plsc.VectorSubcoreMesh(core_axis_name="c", subcore_axis_name="s",
                                num_cores=2, num_subcores=16),
    compiler_params=pltpu.CompilerParams(needs_layout_passes=False,
                                         use_tc_tiling_on_sc=False),
    scratch_types=(
        [pltpu.VMEM((8, LANES), jnp.int32)] * 6
        + [pltpu.VMEM((CH, 8), _F32)] * 4
        + [pltpu.VMEM((CH,), _F32)] * 2
        + [pltpu.VMEM((16,), _F32)]
        + [pltpu.VMEM((ZROWS, 8), _F32)]
        + [pltpu.VMEM_SHARED((NP, 8), _F32)]
        + [pltpu.VMEM_SHARED((NP, 8), _F32)]
        + [pltpu.VMEM_SHARED((NP,), _F32)]
        + [pltpu.SemaphoreType.DMA] * 8
    ),
)(_sc_edge_body)


def _cols(acc):
    return jnp.moveaxis(acc.reshape(NR, LANES, 8), 2, 0)


def _assemble_rec(hst, cfeat):
    ones = jnp.ones((1, NR, LANES), _F32)
    parts = [hst[:cfeat], hst[cfeat:cfeat + 1], ones]
    pad = 8 - (cfeat + 2)
    if pad:
        parts.append(jnp.zeros((pad, NR, LANES), _F32))
    return jnp.moveaxis(jnp.concatenate(parts, 0), 0, 2).reshape(NP, 8)


def kernel(x, edge_index, W1, a_src1, a_dst1, b1, W2, a_src2, a_dst2, b2,
           W3, a_src3, a_dst3, b3):
    xp = jnp.pad(x[:, 0], (0, NP - N)).reshape(NR, LANES)
    epad = jnp.full((2, EPAD - E), NP - 1, jnp.int32)
    ei = jnp.concatenate([edge_index, epad], axis=1)
    srcv = ei[0].reshape(EPAD // LANES, LANES)
    dstv = ei[1].reshape(EPAD // LANES, LANES)

    d0 = _tc_call(_d0_body, 1, 3, [(HID + 2, NR, LANES), (NR, LANES)])
    hst1, a1 = d0(xp, W1, a_src1, a_dst1)

    acc1a, acc1b = _sc_edge(_assemble_rec(hst1, HID),
                            hst1[HID + 1].reshape(NP),
                            a1.reshape(NP)[:16], srcv, dstv)

    fin12 = _tc_call(_make_fin_body(HID, HID), 4, 4,
                     [(HID + 2, NR, LANES), (NR, LANES)])
    hst2, a2 = fin12(_cols(acc1a), _cols(acc1b), hst1, a1,
                     W2, a_src2, a_dst2, b1)

    acc2a, acc2b = _sc_edge(_assemble_rec(hst2, HID),
                            hst2[HID + 1].reshape(NP),
                            a2.reshape(NP)[:16], srcv, dstv)

    fin23 = _tc_call(_make_fin_body(HID, 1), 4, 4,
                     [(3, NR, LANES), (NR, LANES)])
    hst3, a3 = fin23(_cols(acc2a), _cols(acc2b), hst2, a2,
                     W3, a_src3, a_dst3, b2)

    acc3a, acc3b = _sc_edge(_assemble_rec(hst3, 1),
                            hst3[2].reshape(NP),
                            a3.reshape(NP)[:16], srcv, dstv)

    d3 = _tc_call(_d3_body, 4, 1, [(NR, LANES)])
    out = d3(_cols(acc3a), _cols(acc3b), hst3, a3, b3)

    return out[0].reshape(NP)[:N][:, None]
